# Initial kernel scaffold; baseline (speedup 1.0000x reference)
import jax
import jax.numpy as jnp
from jax.experimental import pallas as pl


def kernel(q_pts, s_pts, s_feats, neighb_inds, Wd1, bd1, Wd2, bd2, Wg, bg, Wa1, ba1, Wa2, ba2):
    raise NotImplementedError("write your pallas kernel here")


# speedup vs baseline: 4.0388x; 4.0388x over previous
import functools

import jax
import jax.numpy as jnp
from jax import lax
from jax.experimental import pallas as pl
from jax.experimental.pallas import tpu as pltpu
from jax.experimental.pallas import tpu_sc as plsc

RADIUS = 2.5
H = 16
C = 256
G = 8
CPG = C // G
N = 10000

NC = 2
NS = 16
NW = NC * NS

TOTAL_ROWS = N * H
RPW = TOTAL_ROWS // NW
CHUNK = 128
NFULL = RPW // CHUNK
TAIL = RPW - NFULL * CHUNK
IDXC = RPW + 8
PW = 8

BM = 400
NBLK = N // BM


def _fold_body(sf_ref, wg_ref, wd2_ref, bd2_ref, bg_ref,
               sfw_ref, w2g_ref, b2_ref):
    i = pl.program_id(0)
    sfw_ref[...] = jnp.dot(sf_ref[...], wg_ref[...],
                           preferred_element_type=jnp.float32)

    @pl.when(i == 0)
    def _():
        w2g_ref[...] = jnp.dot(wd2_ref[...], wg_ref[...],
                               preferred_element_type=jnp.float32)
        b2_ref[...] = jnp.dot(bd2_ref[...], wg_ref[...],
                              preferred_element_type=jnp.float32) + bg_ref[...]


def _sc_gather_body(sfw_hbm, xs_hbm, ys_hbm, zs_hbm, idx_hbm, nfw_hbm,
                    npts_hbm, idx_v, xs_v, ys_v, zs_v, rows_v, pts_v, sem_r):
    wid = lax.axis_index("s") * NC + lax.axis_index("c")
    base = pl.multiple_of(wid * RPW, 8)
    pltpu.sync_copy(idx_hbm.at[pl.ds(base, IDXC)], idx_v)
    pltpu.sync_copy(xs_hbm, xs_v)
    pltpu.sync_copy(ys_hbm, ys_v)
    pltpu.sync_copy(zs_hbm, zs_v)
    lanes = jnp.arange(16, dtype=jnp.int32)

    def chunk(off, nrows):
        cp = pltpu.async_copy(sfw_hbm.at[idx_v.at[pl.ds(off, nrows)]],
                              rows_v.at[pl.ds(0, nrows)], sem_r)
        for g in range(max(1, nrows // 16)):
            goff = pl.multiple_of(off + g * 16, 8)
            idx16 = idx_v[pl.ds(goff, 16)]
            x = plsc.load_gather(xs_v, [idx16])
            y = plsc.load_gather(ys_v, [idx16])
            z = plsc.load_gather(zs_v, [idx16])
            lrow = lanes + g * 16
            if nrows >= 16:
                plsc.store_scatter(pts_v, [lrow * PW + 0], x)
                plsc.store_scatter(pts_v, [lrow * PW + 1], y)
                plsc.store_scatter(pts_v, [lrow * PW + 2], z)
            else:
                m = lanes < nrows
                plsc.store_scatter(pts_v, [lrow * PW + 0], x, mask=m)
                plsc.store_scatter(pts_v, [lrow * PW + 1], y, mask=m)
                plsc.store_scatter(pts_v, [lrow * PW + 2], z, mask=m)
        cp.wait()
        gofs = pl.multiple_of(base + off, 8)
        pltpu.sync_copy(rows_v.at[pl.ds(0, nrows)],
                        nfw_hbm.at[pl.ds(gofs, nrows)])
        pltpu.sync_copy(pts_v.at[pl.ds(0, nrows * PW)],
                        npts_hbm.at[pl.ds(pl.multiple_of(gofs * PW, 8),
                                          nrows * PW)])

    def body(j, _):
        chunk(pl.multiple_of(j * CHUNK, 8), CHUNK)
        return 0

    lax.fori_loop(0, NFULL, body, 0)
    chunk(NFULL * CHUNK, TAIL)


def _main_body(nfw_ref, npts_ref, q_ref, wd1_ref, bd1_ref, w2g_ref, b2_ref,
               wa1_ref, ba1_ref, wa2_ref, ba2_ref, out_ref):
    scale = 1.0 / RADIUS
    q = q_ref[...]
    wd1 = wd1_ref[...]
    bd1 = bd1_ref[...]
    w2g = w2g_ref[...]
    b2 = b2_ref[...]
    colmask = lax.broadcasted_iota(jnp.int32, (BM, PW), 1) < 3

    def nf_prime(h):
        nb = jnp.where(colmask, (npts_ref[h] - q) * scale, 0.0)
        g = jnp.dot(nb, wd1, preferred_element_type=jnp.float32) + bd1
        g = jnp.where(g >= 0, g, 0.1 * g)
        ge = jnp.dot(g, w2g, preferred_element_type=jnp.float32)
        return nfw_ref[h] + ge + b2

    nf0 = nf_prime(0)
    a = jnp.dot(nf0, wa1_ref[...], preferred_element_type=jnp.float32)
    a = a + ba1_ref[...]
    a = jnp.where(a >= 0, a, 0.1 * a)
    aw = jnp.dot(a, wa2_ref[...], preferred_element_type=jnp.float32)
    aw = aw + ba2_ref[...]

    cols = lax.broadcasted_iota(jnp.int32, (CPG, C), 1) // G
    rows = lax.broadcasted_iota(jnp.int32, (CPG, C), 0)
    exp8 = jnp.where(cols == rows, 1.0, 0.0).astype(jnp.float32)

    acc = jnp.zeros((BM, C), jnp.float32)
    for h in range(H):
        nfh = nf0 if h == 0 else nf_prime(h)
        aw_h = aw[:, h * CPG:(h + 1) * CPG]
        awx = jnp.dot(aw_h, exp8, preferred_element_type=jnp.float32)
        acc = acc + nfh * awx
    out_ref[...] = acc


@jax.jit
def kernel(q_pts, s_pts, s_feats, neighb_inds, Wd1, bd1, Wd2, bd2, Wg, bg,
           Wa1, ba1, Wa2, ba2):
    idx = neighb_inds.astype(jnp.int32)
    idx_t = idx.T.reshape(-1)
    idx_t = jnp.pad(idx_t, (0, 64))
    xs = s_pts[:, 0]
    ys = s_pts[:, 1]
    zs = s_pts[:, 2]
    q_pts_pad = jnp.pad(q_pts, ((0, 0), (0, PW - 3)))
    wd1_pad = jnp.pad(Wd1, ((0, PW - 3), (0, 0)))
    bd1_2 = bd1.reshape(1, C)
    bd2_2 = bd2.reshape(1, C)
    bg_2 = bg.reshape(1, C)
    ba1_2 = ba1.reshape(1, C)
    ba2_2 = ba2.reshape(1, H * CPG)

    ab = 2000
    sfw, w2g, b2 = pl.pallas_call(
        _fold_body,
        grid=(N // ab,),
        in_specs=[
            pl.BlockSpec((ab, C), lambda i: (i, 0)),
            pl.BlockSpec((C, C), lambda i: (0, 0)),
            pl.BlockSpec((C, C), lambda i: (0, 0)),
            pl.BlockSpec((1, C), lambda i: (0, 0)),
            pl.BlockSpec((1, C), lambda i: (0, 0)),
        ],
        out_specs=[
            pl.BlockSpec((ab, C), lambda i: (i, 0)),
            pl.BlockSpec((C, C), lambda i: (0, 0)),
            pl.BlockSpec((1, C), lambda i: (0, 0)),
        ],
        out_shape=[
            jax.ShapeDtypeStruct((N, C), jnp.float32),
            jax.ShapeDtypeStruct((C, C), jnp.float32),
            jax.ShapeDtypeStruct((1, C), jnp.float32),
        ],
    )(s_feats, Wg, Wd2, bd2_2, bg_2)

    mesh = plsc.VectorSubcoreMesh(core_axis_name="c", subcore_axis_name="s")
    gather_fn = pl.kernel(
        _sc_gather_body,
        out_type=[
            jax.ShapeDtypeStruct((TOTAL_ROWS, C), jnp.float32),
            jax.ShapeDtypeStruct((TOTAL_ROWS * PW,), jnp.float32),
        ],
        mesh=mesh,
        scratch_types=[
            pltpu.VMEM((IDXC,), jnp.int32),
            pltpu.VMEM((N,), jnp.float32),
            pltpu.VMEM((N,), jnp.float32),
            pltpu.VMEM((N,), jnp.float32),
            pltpu.VMEM((CHUNK, C), jnp.float32),
            pltpu.VMEM((CHUNK * PW,), jnp.float32),
            pltpu.SemaphoreType.DMA,
        ],
        compiler_params=pltpu.CompilerParams(needs_layout_passes=False),
    )
    nfw_flat, npts_flat = gather_fn(sfw, xs, ys, zs, idx_t)
    nfw = nfw_flat.reshape(H, N, C)
    npts = npts_flat.reshape(H, N, PW)

    out = pl.pallas_call(
        _main_body,
        grid=(NBLK,),
        in_specs=[
            pl.BlockSpec((H, BM, C), lambda i: (0, i, 0)),
            pl.BlockSpec((H, BM, PW), lambda i: (0, i, 0)),
            pl.BlockSpec((BM, PW), lambda i: (i, 0)),
            pl.BlockSpec((PW, C), lambda i: (0, 0)),
            pl.BlockSpec((1, C), lambda i: (0, 0)),
            pl.BlockSpec((C, C), lambda i: (0, 0)),
            pl.BlockSpec((1, C), lambda i: (0, 0)),
            pl.BlockSpec((C, C), lambda i: (0, 0)),
            pl.BlockSpec((1, C), lambda i: (0, 0)),
            pl.BlockSpec((C, H * CPG), lambda i: (0, 0)),
            pl.BlockSpec((1, H * CPG), lambda i: (0, 0)),
        ],
        out_specs=pl.BlockSpec((BM, C), lambda i: (i, 0)),
        out_shape=jax.ShapeDtypeStruct((N, C), jnp.float32),
    )(nfw, npts, q_pts_pad, wd1_pad, bd1_2, w2g, b2, Wa1, ba1_2,
      Wa2, ba2_2)
    return out


# speedup vs baseline: 4.2325x; 1.0480x over previous
import functools

import jax
import jax.numpy as jnp
from jax import lax
from jax.experimental import pallas as pl
from jax.experimental.pallas import tpu as pltpu
from jax.experimental.pallas import tpu_sc as plsc

RADIUS = 2.5
H = 16
C = 256
G = 8
CPG = C // G
N = 10000

NC = 2
NS = 16
NW = NC * NS

TOTAL_ROWS = N * H
RPW = TOTAL_ROWS // NW
CHUNK = 128
NFULL = RPW // CHUNK
TAIL = RPW - NFULL * CHUNK
IDXC = RPW + 8
PW = 8

BM = 400
NBLK = N // BM


def _fold_body(sf_ref, wg_ref, wd2_ref, bd2_ref, bg_ref,
               sfw_ref, w2g_ref, b2_ref):
    i = pl.program_id(0)
    sfw_ref[...] = jnp.dot(sf_ref[...], wg_ref[...],
                           preferred_element_type=jnp.float32)

    @pl.when(i == 0)
    def _():
        w2g_ref[...] = jnp.dot(wd2_ref[...], wg_ref[...],
                               preferred_element_type=jnp.float32)
        b2_ref[...] = jnp.dot(bd2_ref[...], wg_ref[...],
                              preferred_element_type=jnp.float32) + bg_ref[...]


def _sc_gather_body(sfw_hbm, xs_hbm, ys_hbm, zs_hbm, idx_hbm, nfw_hbm,
                    npts_hbm, idx_v, xs_v, ys_v, zs_v, rows_v, pts_v, sem_r):
    wid = lax.axis_index("s") * NC + lax.axis_index("c")
    base = pl.multiple_of(wid * RPW, 8)
    pltpu.sync_copy(idx_hbm.at[pl.ds(base, IDXC)], idx_v)
    pltpu.sync_copy(xs_hbm, xs_v)
    pltpu.sync_copy(ys_hbm, ys_v)
    pltpu.sync_copy(zs_hbm, zs_v)
    lanes = jnp.arange(16, dtype=jnp.int32)

    def chunk(off, nrows):
        cp = pltpu.async_copy(sfw_hbm.at[idx_v.at[pl.ds(off, nrows)]],
                              rows_v.at[pl.ds(0, nrows)], sem_r)
        for g in range(max(1, nrows // 16)):
            goff = pl.multiple_of(off + g * 16, 8)
            idx16 = idx_v[pl.ds(goff, 16)]
            x = plsc.load_gather(xs_v, [idx16])
            y = plsc.load_gather(ys_v, [idx16])
            z = plsc.load_gather(zs_v, [idx16])
            lrow = lanes + g * 16
            if nrows >= 16:
                plsc.store_scatter(pts_v, [lrow * PW + 0], x)
                plsc.store_scatter(pts_v, [lrow * PW + 1], y)
                plsc.store_scatter(pts_v, [lrow * PW + 2], z)
            else:
                m = lanes < nrows
                plsc.store_scatter(pts_v, [lrow * PW + 0], x, mask=m)
                plsc.store_scatter(pts_v, [lrow * PW + 1], y, mask=m)
                plsc.store_scatter(pts_v, [lrow * PW + 2], z, mask=m)
        cp.wait()
        gofs = pl.multiple_of(base + off, 8)
        pltpu.sync_copy(rows_v.at[pl.ds(0, nrows)],
                        nfw_hbm.at[pl.ds(gofs, nrows)])
        pltpu.sync_copy(pts_v.at[pl.ds(0, nrows * PW)],
                        npts_hbm.at[pl.ds(pl.multiple_of(gofs * PW, 8),
                                          nrows * PW)])

    def body(j, _):
        chunk(pl.multiple_of(j * CHUNK, 8), CHUNK)
        return 0

    lax.fori_loop(0, NFULL, body, 0)
    chunk(NFULL * CHUNK, TAIL)


def _main_body(nfw_ref, npts_ref, q_ref, wd1_ref, bd1_ref, w2g_ref, b2_ref,
               wa1_ref, ba1_ref, wa2_ref, ba2_ref, out_ref):
    scale = 1.0 / RADIUS
    q = q_ref[...]
    wd1 = wd1_ref[...]
    bd1 = bd1_ref[...]
    w2g = w2g_ref[...]
    b2 = b2_ref[...]
    colmask = lax.broadcasted_iota(jnp.int32, (H * BM, PW), 1) < 3

    npts = npts_ref[...].reshape(H * BM, PW)
    nb = jnp.where(colmask, (npts - jnp.broadcast_to(q, (H, BM, PW))
                             .reshape(H * BM, PW)) * scale, 0.0)
    g = jnp.dot(nb, wd1, preferred_element_type=jnp.float32) + bd1
    g = jnp.where(g >= 0, g, 0.1 * g)
    ge = jnp.dot(g, w2g, preferred_element_type=jnp.float32)
    nfp = nfw_ref[...].reshape(H * BM, C) + ge + b2

    pooled = nfp[0:BM]
    a = jnp.dot(pooled, wa1_ref[...], preferred_element_type=jnp.float32)
    a = a + ba1_ref[...]
    a = jnp.where(a >= 0, a, 0.1 * a)
    aw = jnp.dot(a, wa2_ref[...], preferred_element_type=jnp.float32)
    aw = aw + ba2_ref[...]

    cols = lax.broadcasted_iota(jnp.int32, (CPG, C), 1) // G
    rows = lax.broadcasted_iota(jnp.int32, (CPG, C), 0)
    exp8 = jnp.where(cols == rows, 1.0, 0.0).astype(jnp.float32)

    acc = jnp.zeros((BM, C), jnp.float32)
    for h in range(H):
        aw_h = aw[:, h * CPG:(h + 1) * CPG]
        awx = jnp.dot(aw_h, exp8, preferred_element_type=jnp.float32)
        acc = acc + nfp[h * BM:(h + 1) * BM] * awx
    out_ref[...] = acc


@jax.jit
def kernel(q_pts, s_pts, s_feats, neighb_inds, Wd1, bd1, Wd2, bd2, Wg, bg,
           Wa1, ba1, Wa2, ba2):
    idx = neighb_inds.astype(jnp.int32)
    idx_t = idx.T.reshape(-1)
    idx_t = jnp.pad(idx_t, (0, 64))
    xs = s_pts[:, 0]
    ys = s_pts[:, 1]
    zs = s_pts[:, 2]
    q_pts_pad = jnp.pad(q_pts, ((0, 0), (0, PW - 3)))
    wd1_pad = jnp.pad(Wd1, ((0, PW - 3), (0, 0)))
    bd1_2 = bd1.reshape(1, C)
    bd2_2 = bd2.reshape(1, C)
    bg_2 = bg.reshape(1, C)
    ba1_2 = ba1.reshape(1, C)
    ba2_2 = ba2.reshape(1, H * CPG)

    ab = 2000
    sfw, w2g, b2 = pl.pallas_call(
        _fold_body,
        grid=(N // ab,),
        in_specs=[
            pl.BlockSpec((ab, C), lambda i: (i, 0)),
            pl.BlockSpec((C, C), lambda i: (0, 0)),
            pl.BlockSpec((C, C), lambda i: (0, 0)),
            pl.BlockSpec((1, C), lambda i: (0, 0)),
            pl.BlockSpec((1, C), lambda i: (0, 0)),
        ],
        out_specs=[
            pl.BlockSpec((ab, C), lambda i: (i, 0)),
            pl.BlockSpec((C, C), lambda i: (0, 0)),
            pl.BlockSpec((1, C), lambda i: (0, 0)),
        ],
        out_shape=[
            jax.ShapeDtypeStruct((N, C), jnp.float32),
            jax.ShapeDtypeStruct((C, C), jnp.float32),
            jax.ShapeDtypeStruct((1, C), jnp.float32),
        ],
    )(s_feats, Wg, Wd2, bd2_2, bg_2)

    mesh = plsc.VectorSubcoreMesh(core_axis_name="c", subcore_axis_name="s")
    gather_fn = pl.kernel(
        _sc_gather_body,
        out_type=[
            jax.ShapeDtypeStruct((TOTAL_ROWS, C), jnp.float32),
            jax.ShapeDtypeStruct((TOTAL_ROWS * PW,), jnp.float32),
        ],
        mesh=mesh,
        scratch_types=[
            pltpu.VMEM((IDXC,), jnp.int32),
            pltpu.VMEM((N,), jnp.float32),
            pltpu.VMEM((N,), jnp.float32),
            pltpu.VMEM((N,), jnp.float32),
            pltpu.VMEM((CHUNK, C), jnp.float32),
            pltpu.VMEM((CHUNK * PW,), jnp.float32),
            pltpu.SemaphoreType.DMA,
        ],
        compiler_params=pltpu.CompilerParams(needs_layout_passes=False),
    )
    nfw_flat, npts_flat = gather_fn(sfw, xs, ys, zs, idx_t)
    nfw = nfw_flat.reshape(H, N, C)
    npts = npts_flat.reshape(H, N, PW)

    out = pl.pallas_call(
        _main_body,
        grid=(NBLK,),
        in_specs=[
            pl.BlockSpec((H, BM, C), lambda i: (0, i, 0)),
            pl.BlockSpec((H, BM, PW), lambda i: (0, i, 0)),
            pl.BlockSpec((BM, PW), lambda i: (i, 0)),
            pl.BlockSpec((PW, C), lambda i: (0, 0)),
            pl.BlockSpec((1, C), lambda i: (0, 0)),
            pl.BlockSpec((C, C), lambda i: (0, 0)),
            pl.BlockSpec((1, C), lambda i: (0, 0)),
            pl.BlockSpec((C, C), lambda i: (0, 0)),
            pl.BlockSpec((1, C), lambda i: (0, 0)),
            pl.BlockSpec((C, H * CPG), lambda i: (0, 0)),
            pl.BlockSpec((1, H * CPG), lambda i: (0, 0)),
        ],
        out_specs=pl.BlockSpec((BM, C), lambda i: (i, 0)),
        out_shape=jax.ShapeDtypeStruct((N, C), jnp.float32),
    )(nfw, npts, q_pts_pad, wd1_pad, bd1_2, w2g, b2, Wa1, ba1_2,
      Wa2, ba2_2)
    return out
